# Initial kernel scaffold; baseline (speedup 1.0000x reference)
import jax
import jax.numpy as jnp
from jax.experimental import pallas as pl


def kernel(atomic_number, chirality_type, edge_index, bond_type, bond_direction_type, graph_ids, atom_emb, chirality_emb, bond_emb, dir_emb, W1, b1, W2, b2, gamma, beta):
    raise NotImplementedError("write your pallas kernel here")


# speedup vs baseline: 1.4319x; 1.4319x over previous
import functools

import jax
import jax.numpy as jnp
from jax import lax
from jax.experimental import pallas as pl
from jax.experimental.pallas import tpu as pltpu
from jax.experimental.pallas import tpu_sc as plsc

N = 10000
E = 160000
D = 300
L = 5
G = 256

DS = 128
DP = 384
HID = 640
NP = 10112
RPT = 632
BS = 400
NB = N // BS
EP = 163840
CH = 128
NCS = 80
NCH = 40
ET = 16
DUMMY = N + 8

_PREC = lax.Precision.HIGHEST
_PREC_MLP = lax.Precision.DEFAULT

_f32 = jnp.float32
_i32 = jnp.int32


def _mesh():
  return plsc.VectorSubcoreMesh(
      core_axis_name="c", subcore_axis_name="s", num_cores=2, num_subcores=16)


def _embed_call(an_r, ch_r, emb):
  def body(an_ref, ch_ref, emb_ref, o0_ref, o1_ref, o2_ref):
    an = an_ref[0]
    ch = ch_ref[0]
    it = lax.broadcasted_iota(_i32, (136, BS), 0)
    oh = jnp.logical_or(it == an, it == (ch + 128)).astype(_f32)
    h0 = lax.dot_general(oh, emb_ref[...], (((0,), (0,)), ((), ())),
                         precision=_PREC)
    o0_ref[...] = h0[:, :DS]
    o1_ref[...] = h0[:, DS:2 * DS]
    o2_ref[...] = h0[:, 2 * DS:]

  return pl.pallas_call(
      body,
      grid=(NB,),
      in_specs=[
          pl.BlockSpec((1, 1, BS), lambda i: (i, 0, 0)),
          pl.BlockSpec((1, 1, BS), lambda i: (i, 0, 0)),
          pl.BlockSpec((136, DP), lambda i: (0, 0)),
      ],
      out_specs=[pl.BlockSpec((BS, DS), lambda i: (i, 0))] * 3,
      out_shape=[jax.ShapeDtypeStruct((N, DS), _f32)] * 3,
  )(an_r, ch_r, emb)


def _mlp_call(PA, PB, ha, hb, hc, C2a, C2b, Et, sc, W1p, b1p, W2p, b2p):
  def body(pa_ref, pb_ref, ha_ref, hb_ref, hc_ref, ca_ref, cb_ref, et_ref,
           sc_ref, w1_ref, b1_ref, w2_ref, b2_ref, h2_ref, st_ref):
    i = pl.program_id(0)
    a0 = pa_ref[0] + ha_ref[...]
    a1 = pa_ref[1] + hb_ref[...]
    a2 = pb_ref[0] + pb_ref[1] + hc_ref[...]
    agg = jnp.concatenate([a0, a1, a2], axis=1)
    cc = (ca_ref[0] + ca_ref[1] + cb_ref[0] + cb_ref[1])[:, :ET]
    agg = agg + jnp.dot(cc, et_ref[...], precision=_PREC) + sc_ref[...]
    hid = jnp.maximum(
        jnp.dot(agg, w1_ref[...], precision=_PREC_MLP) + b1_ref[0], 0.0)
    h2 = jnp.dot(hid, w2_ref[...], precision=_PREC_MLP) + b2_ref[0]
    h2_ref[...] = h2
    s = jnp.sum(h2, axis=0, keepdims=True)
    q = jnp.sum(h2 * h2, axis=0, keepdims=True)
    sq = jnp.concatenate([s, q, jnp.zeros((6, DP), _f32)], axis=0)

    @pl.when(i == 0)
    def _():
      st_ref[...] = sq

    @pl.when(i > 0)
    def _():
      st_ref[...] = st_ref[...] + sq

  return pl.pallas_call(
      body,
      grid=(NB,),
      in_specs=[
          pl.BlockSpec((2, BS, DS), lambda i: (0, i, 0)),
          pl.BlockSpec((2, BS, DS), lambda i: (0, i, 0)),
          pl.BlockSpec((BS, DS), lambda i: (i, 0)),
          pl.BlockSpec((BS, DS), lambda i: (i, 0)),
          pl.BlockSpec((BS, DS), lambda i: (i, 0)),
          pl.BlockSpec((2, BS, DS), lambda i: (0, i, 0)),
          pl.BlockSpec((2, BS, DS), lambda i: (0, i, 0)),
          pl.BlockSpec((ET, DP), lambda i: (0, 0)),
          pl.BlockSpec((1, DP), lambda i: (0, 0)),
          pl.BlockSpec((DP, HID), lambda i: (0, 0)),
          pl.BlockSpec((1, HID), lambda i: (0, 0)),
          pl.BlockSpec((HID, DP), lambda i: (0, 0)),
          pl.BlockSpec((1, DP), lambda i: (0, 0)),
      ],
      out_specs=[
          pl.BlockSpec((BS, DP), lambda i: (i, 0)),
          pl.BlockSpec((8, DP), lambda i: (0, 0)),
      ],
      out_shape=[
          jax.ShapeDtypeStruct((N, DP), _f32),
          jax.ShapeDtypeStruct((8, DP), _f32),
      ],
  )(PA, PB, ha, hb, hc, C2a, C2b, Et, sc, W1p, b1p, W2p, b2p)


def _bn_call(h2, st, gm, bt, relu):
  def body(h2_ref, st_ref, g_ref, b_ref, o0_ref, o1_ref, o2_ref):
    mean = st_ref[0:1] / float(N)
    var = st_ref[1:2] / float(N) - mean * mean
    scale = g_ref[...] * lax.rsqrt(var + 1e-5)
    y = (h2_ref[...] - mean) * scale + b_ref[...]
    if relu:
      y = jnp.maximum(y, 0.0)
    o0_ref[...] = y[:, :DS]
    o1_ref[...] = y[:, DS:2 * DS]
    o2_ref[...] = y[:, 2 * DS:]

  return pl.pallas_call(
      body,
      grid=(NB,),
      in_specs=[
          pl.BlockSpec((BS, DP), lambda i: (i, 0)),
          pl.BlockSpec((8, DP), lambda i: (0, 0)),
          pl.BlockSpec((1, DP), lambda i: (0, 0)),
          pl.BlockSpec((1, DP), lambda i: (0, 0)),
      ],
      out_specs=[pl.BlockSpec((BS, DS), lambda i: (i, 0))] * 3,
      out_shape=[jax.ShapeDtypeStruct((N, DS), _f32)] * 3,
  )(h2, st, gm, bt)


def _read_call(gid_r, ha, hb, hc):
  def body(g_ref, ha_ref, hb_ref, hc_ref, o_ref, acc, cnt):
    i = pl.program_id(0)
    gid = g_ref[0]
    it = lax.broadcasted_iota(_i32, (G, BS), 0)
    oh = (it == gid).astype(_f32)
    h = jnp.concatenate([ha_ref[...], hb_ref[...], hc_ref[...]], axis=1)
    ps = lax.dot_general(oh, h, (((1,), (0,)), ((), ())), precision=_PREC)
    pc = jnp.sum(oh, axis=1, keepdims=True)

    @pl.when(i == 0)
    def _():
      acc[...] = ps
      cnt[...] = pc

    @pl.when(i > 0)
    def _():
      acc[...] = acc[...] + ps
      cnt[...] = cnt[...] + pc

    @pl.when(i == NB - 1)
    def _():
      o_ref[...] = acc[...] / jnp.maximum(cnt[...], 1.0)

  return pl.pallas_call(
      body,
      grid=(NB,),
      in_specs=[
          pl.BlockSpec((1, 1, BS), lambda i: (i, 0, 0)),
          pl.BlockSpec((BS, DS), lambda i: (i, 0)),
          pl.BlockSpec((BS, DS), lambda i: (i, 0)),
          pl.BlockSpec((BS, DS), lambda i: (i, 0)),
      ],
      out_specs=pl.BlockSpec((G, DP), lambda i: (0, 0)),
      out_shape=jax.ShapeDtypeStruct((G, DP), _f32),
      scratch_shapes=[
          pltpu.VMEM((G, DP), _f32),
          pltpu.VMEM((G, 1), _f32),
      ],
  )(gid_r, ha, hb, hc)


def _gather_scatter_loop(table, src_v, dst_v, bufa, bufb, acc, sema, semb,
                         nchunks):
  pltpu.async_copy(table.at[src_v.at[0]], bufa, sema)

  def pair(jj, carry):
    j0 = jj * 2
    pltpu.async_copy(table.at[src_v.at[j0 + 1]], bufb, semb)
    pltpu.make_async_copy(table.at[src_v.at[j0]], bufa, sema).wait()
    pltpu.sync_copy(bufa, acc.at[dst_v.at[j0]], add=True)

    @pl.when(jj < (nchunks // 2 - 1))
    def _():
      pltpu.async_copy(table.at[src_v.at[j0 + 2]], bufa, sema)

    pltpu.make_async_copy(table.at[src_v.at[j0 + 1]], bufb, semb).wait()
    pltpu.sync_copy(bufb, acc.at[dst_v.at[j0 + 1]], add=True)
    return carry

  lax.fori_loop(0, nchunks // 2, pair, 0)


def _spmm_a_call(ha, hb, src_s, dst_s, zr):
  @functools.partial(
      pl.kernel,
      out_type=jax.ShapeDtypeStruct((2, NP, DS), _f32),
      mesh=_mesh(),
      scratch_types=[
          pltpu.VMEM((NCS // 2, CH), _i32),
          pltpu.VMEM((NCS // 2, CH), _i32),
          pltpu.VMEM((CH, DS), _f32),
          pltpu.VMEM((CH, DS), _f32),
          pltpu.VMEM_SHARED((NP, DS), _f32),
          pltpu.SemaphoreType.DMA,
          pltpu.SemaphoreType.DMA,
      ],
  )
  def k(ha_hbm, hb_hbm, src_hbm, dst_hbm, z_hbm, out_hbm,
        src_v, dst_v, bufa, bufb, acc, sema, semb):
    cid = lax.axis_index("c")
    sid = lax.axis_index("s")
    half = NCS // 2
    pltpu.sync_copy(z_hbm, acc.at[pl.ds(sid * RPT, RPT)])
    plsc.subcore_barrier()

    def run(table):
      for p in range(2):
        pltpu.sync_copy(src_hbm.at[sid, pl.ds(p * half, half)], src_v)
        pltpu.sync_copy(dst_hbm.at[sid, pl.ds(p * half, half)], dst_v)
        _gather_scatter_loop(table, src_v, dst_v, bufa, bufb, acc, sema, semb,
                             half)

    @pl.when(cid == 0)
    def _():
      run(ha_hbm)

    @pl.when(cid == 1)
    def _():
      run(hb_hbm)

    plsc.subcore_barrier()
    pltpu.sync_copy(acc.at[pl.ds(sid * RPT, RPT)],
                    out_hbm.at[cid, pl.ds(sid * RPT, RPT)])

  return k(ha, hb, src_s, dst_s, zr)


def _spmm_b_call(hc, src_s, dst_s, zr):
  @functools.partial(
      pl.kernel,
      out_type=jax.ShapeDtypeStruct((2, NP, DS), _f32),
      mesh=_mesh(),
      scratch_types=[
          pltpu.VMEM((NCS // 2, CH), _i32),
          pltpu.VMEM((NCS // 2, CH), _i32),
          pltpu.VMEM((CH, DS), _f32),
          pltpu.VMEM((CH, DS), _f32),
          pltpu.VMEM_SHARED((NP, DS), _f32),
          pltpu.SemaphoreType.DMA,
          pltpu.SemaphoreType.DMA,
      ],
  )
  def k(hc_hbm, src_hbm, dst_hbm, z_hbm, out_hbm,
        src_v, dst_v, bufa, bufb, acc, sema, semb):
    cid = lax.axis_index("c")
    sid = lax.axis_index("s")
    half = NCS // 2
    pltpu.sync_copy(src_hbm.at[sid, pl.ds(cid * half, half)], src_v)
    pltpu.sync_copy(dst_hbm.at[sid, pl.ds(cid * half, half)], dst_v)
    pltpu.sync_copy(z_hbm, acc.at[pl.ds(sid * RPT, RPT)])
    plsc.subcore_barrier()
    _gather_scatter_loop(hc_hbm, src_v, dst_v, bufa, bufb, acc, sema, semb,
                         half)
    plsc.subcore_barrier()
    pltpu.sync_copy(acc.at[pl.ds(sid * RPT, RPT)],
                    out_hbm.at[cid, pl.ds(sid * RPT, RPT)])

  return k(hc, src_s, dst_s, zr)


def kernel(atomic_number, chirality_type, edge_index, bond_type,
           bond_direction_type, graph_ids, atom_emb, chirality_emb,
           bond_emb, dir_emb, W1, b1, W2, b2, gamma, beta):
  an = atomic_number.astype(_i32)
  ch = chirality_type.astype(_i32)
  src = edge_index[0].astype(_i32)
  dst = edge_index[1].astype(_i32)
  bt = bond_type.astype(_i32)
  bdt = bond_direction_type.astype(_i32)
  gid = graph_ids.astype(_i32)

  emb = (jnp.zeros((136, DP), _f32)
         .at[:120, :D].set(atom_emb)
         .at[128:131, :D].set(chirality_emb))
  Et = (jnp.zeros((L, ET, DP), _f32)
        .at[:, :6, :D].set(bond_emb)
        .at[:, 6:9, :D].set(dir_emb))
  W1p = jnp.zeros((L, DP, HID), _f32).at[:, :D, :2 * D].set(W1)
  selfc = (jnp.zeros((L, 1, DP), _f32)
           .at[:, 0, :D].set(bond_emb[:, 4, :] + dir_emb[:, 0, :]))
  b1p = jnp.zeros((L, HID), _f32).at[:, :2 * D].set(b1)
  W2p = jnp.zeros((L, HID, DP), _f32).at[:, :2 * D, :D].set(W2)
  b2p = jnp.zeros((L, DP), _f32).at[:, :D].set(b2)
  gmp = jnp.zeros((L, DP), _f32).at[:, :D].set(gamma)
  btp = jnp.zeros((L, DP), _f32).at[:, :D].set(beta)

  pad = EP - E
  srcp = jnp.concatenate([src, jnp.zeros((pad,), _i32)])
  dstp = jnp.concatenate([dst, jnp.full((pad,), DUMMY, _i32)])
  btp_e = jnp.concatenate([bt, jnp.zeros((pad,), _i32)])
  bd6_e = jnp.concatenate([bdt + 6, jnp.full((pad,), 6, _i32)])
  src_s = srcp.reshape(16, NCS, CH)
  dst_s = dstp.reshape(16, NCS, CH)
  bt_s = btp_e.reshape(16, NCS, CH)
  bd6_s = bd6_e.reshape(16, NCS, CH)
  an_r = an.reshape(NB, 1, BS)
  ch_r = ch.reshape(NB, 1, BS)
  gid_r = gid.reshape(NB, 1, BS)
  zr = jnp.zeros((RPT, DS), _f32)
  T = jnp.eye(ET, DS, dtype=_f32)

  ha, hb, hc = _embed_call(an_r, ch_r, emb)
  C2a = _spmm_b_call(T, bt_s, dst_s, zr)
  C2b = _spmm_b_call(T, bd6_s, dst_s, zr)
  for l in range(L):
    PA = _spmm_a_call(ha, hb, src_s, dst_s, zr)
    PB = _spmm_b_call(hc, src_s, dst_s, zr)
    h2, st = _mlp_call(PA, PB, ha, hb, hc, C2a, C2b, Et[l], selfc[l], W1p[l],
                       b1p[l].reshape(1, HID), W2p[l], b2p[l].reshape(1, DP))
    ha, hb, hc = _bn_call(h2, st, gmp[l].reshape(1, DP), btp[l].reshape(1, DP),
                          l < L - 1)
  out = _read_call(gid_r, ha, hb, hc)
  return out[:, :D]


# speedup vs baseline: 1.4324x; 1.0004x over previous
import functools

import jax
import jax.numpy as jnp
from jax import lax
from jax.experimental import pallas as pl
from jax.experimental.pallas import tpu as pltpu
from jax.experimental.pallas import tpu_sc as plsc

N = 10000
E = 160000
D = 300
L = 5
G = 256

DS = 128
DP = 384
HID = 640
NP = 10112
RPT = 632
BS = 400
NB = N // BS
EP = 163840
CH = 128
NCS = 80
ET = 16
DUMMY = N + 8

_PREC = lax.Precision.HIGHEST
_PREC_MLP = lax.Precision.DEFAULT

_f32 = jnp.float32
_i32 = jnp.int32


def _mesh():
  return plsc.VectorSubcoreMesh(
      core_axis_name="c", subcore_axis_name="s", num_cores=2, num_subcores=16)


def _embed_call(an_r, ch_r, emb):
  def body(an_ref, ch_ref, emb_ref, o0_ref, o1_ref, o2_ref):
    an = an_ref[0]
    ch = ch_ref[0]
    it = lax.broadcasted_iota(_i32, (136, BS), 0)
    oh = jnp.logical_or(it == an, it == (ch + 128)).astype(_f32)
    h0 = lax.dot_general(oh, emb_ref[...], (((0,), (0,)), ((), ())),
                         precision=_PREC)
    o0_ref[...] = h0[:, :DS]
    o1_ref[...] = h0[:, DS:2 * DS]
    o2_ref[...] = h0[:, 2 * DS:]

  return pl.pallas_call(
      body,
      grid=(NB,),
      in_specs=[
          pl.BlockSpec((1, 1, BS), lambda i: (i, 0, 0)),
          pl.BlockSpec((1, 1, BS), lambda i: (i, 0, 0)),
          pl.BlockSpec((136, DP), lambda i: (0, 0)),
      ],
      out_specs=[pl.BlockSpec((BS, DS), lambda i: (i, 0))] * 3,
      out_shape=[jax.ShapeDtypeStruct((N, DS), _f32)] * 3,
  )(an_r, ch_r, emb)


def _mlp_call(PA, PB, ha, hb, hc, C2a, C2b, Et, sc, W1p, b1p, W2p, b2p):
  def body(pa_ref, pb_ref, ha_ref, hb_ref, hc_ref, ca_ref, cb_ref, et_ref,
           sc_ref, w1_ref, b1_ref, w2_ref, b2_ref, h2_ref, st_ref):
    i = pl.program_id(0)
    a0 = pa_ref[0] + ha_ref[...]
    a1 = pa_ref[1] + hb_ref[...]
    a2 = pb_ref[0] + pb_ref[1] + hc_ref[...]
    agg = jnp.concatenate([a0, a1, a2], axis=1)
    cc = (ca_ref[0] + ca_ref[1] + cb_ref[0] + cb_ref[1])[:, :ET]
    agg = agg + jnp.dot(cc, et_ref[...], precision=_PREC) + sc_ref[...]
    hid = jnp.maximum(
        jnp.dot(agg, w1_ref[...], precision=_PREC_MLP) + b1_ref[0], 0.0)
    h2 = jnp.dot(hid, w2_ref[...], precision=_PREC_MLP) + b2_ref[0]
    h2_ref[...] = h2
    s = jnp.sum(h2, axis=0, keepdims=True)
    q = jnp.sum(h2 * h2, axis=0, keepdims=True)
    sq = jnp.concatenate([s, q, jnp.zeros((6, DP), _f32)], axis=0)

    @pl.when(i == 0)
    def _():
      st_ref[...] = sq

    @pl.when(i > 0)
    def _():
      st_ref[...] = st_ref[...] + sq

  return pl.pallas_call(
      body,
      grid=(NB,),
      in_specs=[
          pl.BlockSpec((2, BS, DS), lambda i: (0, i, 0)),
          pl.BlockSpec((2, BS, DS), lambda i: (0, i, 0)),
          pl.BlockSpec((BS, DS), lambda i: (i, 0)),
          pl.BlockSpec((BS, DS), lambda i: (i, 0)),
          pl.BlockSpec((BS, DS), lambda i: (i, 0)),
          pl.BlockSpec((2, BS, DS), lambda i: (0, i, 0)),
          pl.BlockSpec((2, BS, DS), lambda i: (0, i, 0)),
          pl.BlockSpec((ET, DP), lambda i: (0, 0)),
          pl.BlockSpec((1, DP), lambda i: (0, 0)),
          pl.BlockSpec((DP, HID), lambda i: (0, 0)),
          pl.BlockSpec((1, HID), lambda i: (0, 0)),
          pl.BlockSpec((HID, DP), lambda i: (0, 0)),
          pl.BlockSpec((1, DP), lambda i: (0, 0)),
      ],
      out_specs=[
          pl.BlockSpec((BS, DP), lambda i: (i, 0)),
          pl.BlockSpec((8, DP), lambda i: (0, 0)),
      ],
      out_shape=[
          jax.ShapeDtypeStruct((N, DP), _f32),
          jax.ShapeDtypeStruct((8, DP), _f32),
      ],
  )(PA, PB, ha, hb, hc, C2a, C2b, Et, sc, W1p, b1p, W2p, b2p)


def _bn_call(h2, st, gm, bt, relu):
  def body(h2_ref, st_ref, g_ref, b_ref, o0_ref, o1_ref, o2_ref):
    mean = st_ref[0:1] / float(N)
    var = st_ref[1:2] / float(N) - mean * mean
    scale = g_ref[...] * lax.rsqrt(var + 1e-5)
    y = (h2_ref[...] - mean) * scale + b_ref[...]
    if relu:
      y = jnp.maximum(y, 0.0)
    o0_ref[...] = y[:, :DS]
    o1_ref[...] = y[:, DS:2 * DS]
    o2_ref[...] = y[:, 2 * DS:]

  return pl.pallas_call(
      body,
      grid=(NB,),
      in_specs=[
          pl.BlockSpec((BS, DP), lambda i: (i, 0)),
          pl.BlockSpec((8, DP), lambda i: (0, 0)),
          pl.BlockSpec((1, DP), lambda i: (0, 0)),
          pl.BlockSpec((1, DP), lambda i: (0, 0)),
      ],
      out_specs=[pl.BlockSpec((BS, DS), lambda i: (i, 0))] * 3,
      out_shape=[jax.ShapeDtypeStruct((N, DS), _f32)] * 3,
  )(h2, st, gm, bt)


def _read_call(gid_r, ha, hb, hc):
  def body(g_ref, ha_ref, hb_ref, hc_ref, o_ref, acc, cnt):
    i = pl.program_id(0)
    gid = g_ref[0]
    it = lax.broadcasted_iota(_i32, (G, BS), 0)
    oh = (it == gid).astype(_f32)
    h = jnp.concatenate([ha_ref[...], hb_ref[...], hc_ref[...]], axis=1)
    ps = lax.dot_general(oh, h, (((1,), (0,)), ((), ())), precision=_PREC)
    pc = jnp.sum(oh, axis=1, keepdims=True)

    @pl.when(i == 0)
    def _():
      acc[...] = ps
      cnt[...] = pc

    @pl.when(i > 0)
    def _():
      acc[...] = acc[...] + ps
      cnt[...] = cnt[...] + pc

    @pl.when(i == NB - 1)
    def _():
      o_ref[...] = acc[...] / jnp.maximum(cnt[...], 1.0)

  return pl.pallas_call(
      body,
      grid=(NB,),
      in_specs=[
          pl.BlockSpec((1, 1, BS), lambda i: (i, 0, 0)),
          pl.BlockSpec((BS, DS), lambda i: (i, 0)),
          pl.BlockSpec((BS, DS), lambda i: (i, 0)),
          pl.BlockSpec((BS, DS), lambda i: (i, 0)),
      ],
      out_specs=pl.BlockSpec((G, DP), lambda i: (0, 0)),
      out_shape=jax.ShapeDtypeStruct((G, DP), _f32),
      scratch_shapes=[
          pltpu.VMEM((G, DP), _f32),
          pltpu.VMEM((G, 1), _f32),
      ],
  )(gid_r, ha, hb, hc)


def _gather_scatter_loop(table, src_v, dst_v, bufa, bufb, acc, sema, semb,
                         nchunks):
  pltpu.async_copy(table.at[src_v.at[0]], bufa, sema)

  def pair(jj, carry):
    j0 = jj * 2
    pltpu.async_copy(table.at[src_v.at[j0 + 1]], bufb, semb)
    pltpu.make_async_copy(table.at[src_v.at[j0]], bufa, sema).wait()
    pltpu.sync_copy(bufa, acc.at[dst_v.at[j0]], add=True)

    @pl.when(jj < (nchunks // 2 - 1))
    def _():
      pltpu.async_copy(table.at[src_v.at[j0 + 2]], bufa, sema)

    pltpu.make_async_copy(table.at[src_v.at[j0 + 1]], bufb, semb).wait()
    pltpu.sync_copy(bufb, acc.at[dst_v.at[j0 + 1]], add=True)
    return carry

  lax.fori_loop(0, nchunks // 2, pair, 0)


def _spmm_a_call(ha, hb, src_s, dst_s, zr):
  @functools.partial(
      pl.kernel,
      out_type=jax.ShapeDtypeStruct((2, NP, DS), _f32),
      mesh=_mesh(),
      scratch_types=[
          pltpu.VMEM((NCS // 2, CH), _i32),
          pltpu.VMEM((NCS // 2, CH), _i32),
          pltpu.VMEM((CH, DS), _f32),
          pltpu.VMEM((CH, DS), _f32),
          pltpu.VMEM_SHARED((NP, DS), _f32),
          pltpu.SemaphoreType.DMA,
          pltpu.SemaphoreType.DMA,
      ],
  )
  def k(ha_hbm, hb_hbm, src_hbm, dst_hbm, z_hbm, out_hbm,
        src_v, dst_v, bufa, bufb, acc, sema, semb):
    cid = lax.axis_index("c")
    sid = lax.axis_index("s")
    half = NCS // 2
    pltpu.sync_copy(z_hbm, acc.at[pl.ds(sid * RPT, RPT)])
    plsc.subcore_barrier()

    def run(table):
      for p in range(2):
        pltpu.sync_copy(src_hbm.at[sid, pl.ds(p * half, half)], src_v)
        pltpu.sync_copy(dst_hbm.at[sid, pl.ds(p * half, half)], dst_v)
        _gather_scatter_loop(table, src_v, dst_v, bufa, bufb, acc, sema, semb,
                             half)

    @pl.when(cid == 0)
    def _():
      run(ha_hbm)

    @pl.when(cid == 1)
    def _():
      run(hb_hbm)

    plsc.subcore_barrier()
    pltpu.sync_copy(acc.at[pl.ds(sid * RPT, RPT)],
                    out_hbm.at[cid, pl.ds(sid * RPT, RPT)])

  return k(ha, hb, src_s, dst_s, zr)


def _spmm_b_call(hc, src_s, dst_s, zr):
  @functools.partial(
      pl.kernel,
      out_type=jax.ShapeDtypeStruct((2, NP, DS), _f32),
      mesh=_mesh(),
      scratch_types=[
          pltpu.VMEM((NCS // 2, CH), _i32),
          pltpu.VMEM((NCS // 2, CH), _i32),
          pltpu.VMEM((CH, DS), _f32),
          pltpu.VMEM((CH, DS), _f32),
          pltpu.VMEM_SHARED((NP, DS), _f32),
          pltpu.SemaphoreType.DMA,
          pltpu.SemaphoreType.DMA,
      ],
  )
  def k(hc_hbm, src_hbm, dst_hbm, z_hbm, out_hbm,
        src_v, dst_v, bufa, bufb, acc, sema, semb):
    cid = lax.axis_index("c")
    sid = lax.axis_index("s")
    half = NCS // 2
    pltpu.sync_copy(src_hbm.at[sid, pl.ds(cid * half, half)], src_v)
    pltpu.sync_copy(dst_hbm.at[sid, pl.ds(cid * half, half)], dst_v)
    pltpu.sync_copy(z_hbm, acc.at[pl.ds(sid * RPT, RPT)])
    plsc.subcore_barrier()
    _gather_scatter_loop(hc_hbm, src_v, dst_v, bufa, bufb, acc, sema, semb,
                         half)
    plsc.subcore_barrier()
    pltpu.sync_copy(acc.at[pl.ds(sid * RPT, RPT)],
                    out_hbm.at[cid, pl.ds(sid * RPT, RPT)])

  return k(hc, src_s, dst_s, zr)


def kernel(atomic_number, chirality_type, edge_index, bond_type,
           bond_direction_type, graph_ids, atom_emb, chirality_emb,
           bond_emb, dir_emb, W1, b1, W2, b2, gamma, beta):
  an = atomic_number.astype(_i32)
  ch = chirality_type.astype(_i32)
  src = edge_index[0].astype(_i32)
  dst = edge_index[1].astype(_i32)
  bt = bond_type.astype(_i32)
  bdt = bond_direction_type.astype(_i32)
  gid = graph_ids.astype(_i32)

  emb = (jnp.zeros((136, DP), _f32)
         .at[:120, :D].set(atom_emb)
         .at[128:131, :D].set(chirality_emb))
  Et = (jnp.zeros((L, ET, DP), _f32)
        .at[:, :6, :D].set(bond_emb)
        .at[:, 6:9, :D].set(dir_emb))
  W1p = jnp.zeros((L, DP, HID), _f32).at[:, :D, :2 * D].set(W1)
  selfc = (jnp.zeros((L, 1, DP), _f32)
           .at[:, 0, :D].set(bond_emb[:, 4, :] + dir_emb[:, 0, :]))
  b1p = jnp.zeros((L, HID), _f32).at[:, :2 * D].set(b1)
  W2p = jnp.zeros((L, HID, DP), _f32).at[:, :2 * D, :D].set(W2)
  b2p = jnp.zeros((L, DP), _f32).at[:, :D].set(b2)
  gmp = jnp.zeros((L, DP), _f32).at[:, :D].set(gamma)
  btp = jnp.zeros((L, DP), _f32).at[:, :D].set(beta)

  pad = EP - E
  srcp = jnp.concatenate([src, jnp.zeros((pad,), _i32)])
  dstp = jnp.concatenate([dst, jnp.full((pad,), DUMMY, _i32)])
  btp_e = jnp.concatenate([bt, jnp.zeros((pad,), _i32)])
  bd6_e = jnp.concatenate([bdt + 6, jnp.full((pad,), 6, _i32)])
  src_s = srcp.reshape(16, NCS, CH)
  dst_s = dstp.reshape(16, NCS, CH)
  bt_s = btp_e.reshape(16, NCS, CH)
  bd6_s = bd6_e.reshape(16, NCS, CH)
  an_r = an.reshape(NB, 1, BS)
  ch_r = ch.reshape(NB, 1, BS)
  gid_r = gid.reshape(NB, 1, BS)
  zr = jnp.zeros((RPT, DS), _f32)
  T = jnp.eye(ET, DS, dtype=_f32)

  ha, hb, hc = _embed_call(an_r, ch_r, emb)
  C2a = _spmm_b_call(T, bt_s, dst_s, zr)
  C2b = _spmm_b_call(T, bd6_s, dst_s, zr)
  for l in range(L):
    PA = _spmm_a_call(ha, hb, src_s, dst_s, zr)
    PB = _spmm_b_call(hc, src_s, dst_s, zr)
    h2, st = _mlp_call(PA, PB, ha, hb, hc, C2a, C2b, Et[l], selfc[l], W1p[l],
                       b1p[l].reshape(1, HID), W2p[l], b2p[l].reshape(1, DP))
    ha, hb, hc = _bn_call(h2, st, gmp[l].reshape(1, DP), btp[l].reshape(1, DP),
                          l < L - 1)
  out = _read_call(gid_r, ha, hb, hc)
  return out[:, :D]


# speedup vs baseline: 2.8661x; 2.0009x over previous
import functools

import jax
import jax.numpy as jnp
from jax import lax
from jax.experimental import pallas as pl
from jax.experimental.pallas import tpu as pltpu
from jax.experimental.pallas import tpu_sc as plsc

N = 10000
E = 160000
D = 300
L = 5
G = 256

DS = 128
DP = 384
HID = 640
NP = 10112
RPT = 632
BS = 400
NB = N // BS
EP = 163840
CH = 128
NCS = 80
ET = 16
DUMMY = N + 8

_PREC = lax.Precision.HIGHEST
_PREC_MLP = lax.Precision.DEFAULT

_f32 = jnp.float32
_i32 = jnp.int32


def _mesh():
  return plsc.VectorSubcoreMesh(
      core_axis_name="c", subcore_axis_name="s", num_cores=2, num_subcores=16)


def _embed_call(an_r, ch_r, emb):
  def body(an_ref, ch_ref, emb_ref, o0_ref, o1_ref, o2_ref):
    an = an_ref[0]
    ch = ch_ref[0]
    it = lax.broadcasted_iota(_i32, (136, BS), 0)
    oh = jnp.logical_or(it == an, it == (ch + 128)).astype(_f32)
    h0 = lax.dot_general(oh, emb_ref[...], (((0,), (0,)), ((), ())),
                         precision=_PREC)
    o0_ref[...] = h0[:, :DS]
    o1_ref[...] = h0[:, DS:2 * DS]
    o2_ref[...] = h0[:, 2 * DS:]

  return pl.pallas_call(
      body,
      grid=(NB,),
      in_specs=[
          pl.BlockSpec((1, 1, BS), lambda i: (i, 0, 0)),
          pl.BlockSpec((1, 1, BS), lambda i: (i, 0, 0)),
          pl.BlockSpec((136, DP), lambda i: (0, 0)),
      ],
      out_specs=[pl.BlockSpec((BS, DS), lambda i: (i, 0))] * 3,
      out_shape=[jax.ShapeDtypeStruct((N, DS), _f32)] * 3,
  )(an_r, ch_r, emb)


def _mlp_call(PA, PB, ha, hb, hc, C2a, C2b, Et, sc, W1p, b1p, W2p, b2p):
  def body(pa_ref, pb_ref, ha_ref, hb_ref, hc_ref, ca_ref, cb_ref, et_ref,
           sc_ref, w1_ref, b1_ref, w2_ref, b2_ref, h2_ref, st_ref):
    i = pl.program_id(0)
    a0 = pa_ref[0] + ha_ref[...]
    a1 = pa_ref[1] + hb_ref[...]
    a2 = pb_ref[0] + pb_ref[1] + hc_ref[...]
    agg = jnp.concatenate([a0, a1, a2], axis=1)
    cc = (ca_ref[0] + ca_ref[1] + cb_ref[0] + cb_ref[1])[:, :ET]
    agg = agg + jnp.dot(cc, et_ref[...], precision=_PREC) + sc_ref[...]
    hid = jnp.maximum(
        jnp.dot(agg, w1_ref[...], precision=_PREC_MLP) + b1_ref[0], 0.0)
    h2 = jnp.dot(hid, w2_ref[...], precision=_PREC_MLP) + b2_ref[0]
    h2_ref[...] = h2
    s = jnp.sum(h2, axis=0, keepdims=True)
    q = jnp.sum(h2 * h2, axis=0, keepdims=True)
    sq = jnp.concatenate([s, q, jnp.zeros((6, DP), _f32)], axis=0)

    @pl.when(i == 0)
    def _():
      st_ref[...] = sq

    @pl.when(i > 0)
    def _():
      st_ref[...] = st_ref[...] + sq

  return pl.pallas_call(
      body,
      grid=(NB,),
      in_specs=[
          pl.BlockSpec((2, BS, DS), lambda i: (0, i, 0)),
          pl.BlockSpec((2, BS, DS), lambda i: (0, i, 0)),
          pl.BlockSpec((BS, DS), lambda i: (i, 0)),
          pl.BlockSpec((BS, DS), lambda i: (i, 0)),
          pl.BlockSpec((BS, DS), lambda i: (i, 0)),
          pl.BlockSpec((2, BS, DS), lambda i: (0, i, 0)),
          pl.BlockSpec((2, BS, DS), lambda i: (0, i, 0)),
          pl.BlockSpec((ET, DP), lambda i: (0, 0)),
          pl.BlockSpec((1, DP), lambda i: (0, 0)),
          pl.BlockSpec((DP, HID), lambda i: (0, 0)),
          pl.BlockSpec((1, HID), lambda i: (0, 0)),
          pl.BlockSpec((HID, DP), lambda i: (0, 0)),
          pl.BlockSpec((1, DP), lambda i: (0, 0)),
      ],
      out_specs=[
          pl.BlockSpec((BS, DP), lambda i: (i, 0)),
          pl.BlockSpec((8, DP), lambda i: (0, 0)),
      ],
      out_shape=[
          jax.ShapeDtypeStruct((N, DP), _f32),
          jax.ShapeDtypeStruct((8, DP), _f32),
      ],
  )(PA, PB, ha, hb, hc, C2a, C2b, Et, sc, W1p, b1p, W2p, b2p)


def _bn_call(h2, st, gm, bt, relu):
  def body(h2_ref, st_ref, g_ref, b_ref, o0_ref, o1_ref, o2_ref):
    mean = st_ref[0:1] / float(N)
    var = st_ref[1:2] / float(N) - mean * mean
    scale = g_ref[...] * lax.rsqrt(var + 1e-5)
    y = (h2_ref[...] - mean) * scale + b_ref[...]
    if relu:
      y = jnp.maximum(y, 0.0)
    o0_ref[...] = y[:, :DS]
    o1_ref[...] = y[:, DS:2 * DS]
    o2_ref[...] = y[:, 2 * DS:]

  return pl.pallas_call(
      body,
      grid=(NB,),
      in_specs=[
          pl.BlockSpec((BS, DP), lambda i: (i, 0)),
          pl.BlockSpec((8, DP), lambda i: (0, 0)),
          pl.BlockSpec((1, DP), lambda i: (0, 0)),
          pl.BlockSpec((1, DP), lambda i: (0, 0)),
      ],
      out_specs=[pl.BlockSpec((BS, DS), lambda i: (i, 0))] * 3,
      out_shape=[jax.ShapeDtypeStruct((N, DS), _f32)] * 3,
  )(h2, st, gm, bt)


def _read_call(gid_r, ha, hb, hc):
  def body(g_ref, ha_ref, hb_ref, hc_ref, o_ref, acc, cnt):
    i = pl.program_id(0)
    gid = g_ref[0]
    it = lax.broadcasted_iota(_i32, (G, BS), 0)
    oh = (it == gid).astype(_f32)
    h = jnp.concatenate([ha_ref[...], hb_ref[...], hc_ref[...]], axis=1)
    ps = lax.dot_general(oh, h, (((1,), (0,)), ((), ())), precision=_PREC)
    pc = jnp.sum(oh, axis=1, keepdims=True)

    @pl.when(i == 0)
    def _():
      acc[...] = ps
      cnt[...] = pc

    @pl.when(i > 0)
    def _():
      acc[...] = acc[...] + ps
      cnt[...] = cnt[...] + pc

    @pl.when(i == NB - 1)
    def _():
      o_ref[...] = acc[...] / jnp.maximum(cnt[...], 1.0)

  return pl.pallas_call(
      body,
      grid=(NB,),
      in_specs=[
          pl.BlockSpec((1, 1, BS), lambda i: (i, 0, 0)),
          pl.BlockSpec((BS, DS), lambda i: (i, 0)),
          pl.BlockSpec((BS, DS), lambda i: (i, 0)),
          pl.BlockSpec((BS, DS), lambda i: (i, 0)),
      ],
      out_specs=pl.BlockSpec((G, DP), lambda i: (0, 0)),
      out_shape=jax.ShapeDtypeStruct((G, DP), _f32),
      scratch_shapes=[
          pltpu.VMEM((G, DP), _f32),
          pltpu.VMEM((G, 1), _f32),
      ],
  )(gid_r, ha, hb, hc)


def _gather_scatter_loop(table, src_v, dst_v, bufa, bufb, acc, sema, semb,
                         nchunks):
  pltpu.async_copy(table.at[src_v.at[0]], bufa, sema)

  def pair(jj, carry):
    j0 = jj * 2
    pltpu.async_copy(table.at[src_v.at[j0 + 1]], bufb, semb)
    pltpu.make_async_copy(table.at[src_v.at[j0]], bufa, sema).wait()
    pltpu.sync_copy(bufa, acc.at[dst_v.at[j0]], add=True)

    @pl.when(jj < (nchunks // 2 - 1))
    def _():
      pltpu.async_copy(table.at[src_v.at[j0 + 2]], bufa, sema)

    pltpu.make_async_copy(table.at[src_v.at[j0 + 1]], bufb, semb).wait()
    pltpu.sync_copy(bufb, acc.at[dst_v.at[j0 + 1]], add=True)
    return carry

  lax.fori_loop(0, nchunks // 2, pair, 0)


def _spmm_a_call(ha, hb, src_s, dst_s, zr):
  @functools.partial(
      pl.kernel,
      out_type=jax.ShapeDtypeStruct((2, NP, DS), _f32),
      mesh=_mesh(),
      scratch_types=[
          pltpu.VMEM((NCS // 2, CH), _i32),
          pltpu.VMEM((NCS // 2, CH), _i32),
          pltpu.VMEM((CH, DS), _f32),
          pltpu.VMEM((CH, DS), _f32),
          pltpu.VMEM_SHARED((NP, DS), _f32),
          pltpu.SemaphoreType.DMA,
          pltpu.SemaphoreType.DMA,
      ],
  )
  def k(ha_hbm, hb_hbm, src_hbm, dst_hbm, z_hbm, out_hbm,
        src_v, dst_v, bufa, bufb, acc, sema, semb):
    cid = lax.axis_index("c")
    sid = lax.axis_index("s")
    half = NCS // 2
    pltpu.sync_copy(z_hbm, acc.at[pl.ds(sid * RPT, RPT)])
    plsc.subcore_barrier()

    def run(table):
      for p in range(2):
        pltpu.sync_copy(src_hbm.at[sid, pl.ds(p * half, half)], src_v)
        pltpu.sync_copy(dst_hbm.at[sid, pl.ds(p * half, half)], dst_v)
        _gather_scatter_loop(table, src_v, dst_v, bufa, bufb, acc, sema, semb,
                             half)

    @pl.when(cid == 0)
    def _():
      run(ha_hbm)

    @pl.when(cid == 1)
    def _():
      run(hb_hbm)

    plsc.subcore_barrier()
    pltpu.sync_copy(acc.at[pl.ds(sid * RPT, RPT)],
                    out_hbm.at[cid, pl.ds(sid * RPT, RPT)])

  return k(ha, hb, src_s, dst_s, zr)


def _spmm_b_call(hc, src_s, dst_s, zr, stage_table=False):
  scratch = [
      pltpu.VMEM((NCS // 2, CH), _i32),
      pltpu.VMEM((NCS // 2, CH), _i32),
      pltpu.VMEM((CH, DS), _f32),
      pltpu.VMEM((CH, DS), _f32),
      pltpu.VMEM_SHARED((NP, DS), _f32),
      pltpu.SemaphoreType.DMA,
      pltpu.SemaphoreType.DMA,
  ]
  if stage_table:
    scratch.append(pltpu.VMEM_SHARED((ET, DS), _f32))

  @functools.partial(
      pl.kernel,
      out_type=jax.ShapeDtypeStruct((2, NP, DS), _f32),
      mesh=_mesh(),
      scratch_types=scratch,
  )
  def k(hc_hbm, src_hbm, dst_hbm, z_hbm, out_hbm,
        src_v, dst_v, bufa, bufb, acc, sema, semb, *tsp):
    cid = lax.axis_index("c")
    sid = lax.axis_index("s")
    half = NCS // 2
    pltpu.sync_copy(src_hbm.at[sid, pl.ds(cid * half, half)], src_v)
    pltpu.sync_copy(dst_hbm.at[sid, pl.ds(cid * half, half)], dst_v)
    pltpu.sync_copy(z_hbm, acc.at[pl.ds(sid * RPT, RPT)])
    if stage_table:
      @pl.when(sid == 0)
      def _():
        pltpu.sync_copy(hc_hbm, tsp[0])
    plsc.subcore_barrier()
    table = tsp[0] if stage_table else hc_hbm
    _gather_scatter_loop(table, src_v, dst_v, bufa, bufb, acc, sema, semb,
                         half)
    plsc.subcore_barrier()
    pltpu.sync_copy(acc.at[pl.ds(sid * RPT, RPT)],
                    out_hbm.at[cid, pl.ds(sid * RPT, RPT)])

  return k(hc, src_s, dst_s, zr)


def kernel(atomic_number, chirality_type, edge_index, bond_type,
           bond_direction_type, graph_ids, atom_emb, chirality_emb,
           bond_emb, dir_emb, W1, b1, W2, b2, gamma, beta):
  an = atomic_number.astype(_i32)
  ch = chirality_type.astype(_i32)
  src = edge_index[0].astype(_i32)
  dst = edge_index[1].astype(_i32)
  bt = bond_type.astype(_i32)
  bdt = bond_direction_type.astype(_i32)
  gid = graph_ids.astype(_i32)

  emb = (jnp.zeros((136, DP), _f32)
         .at[:120, :D].set(atom_emb)
         .at[128:131, :D].set(chirality_emb))
  Et = (jnp.zeros((L, ET, DP), _f32)
        .at[:, :6, :D].set(bond_emb)
        .at[:, 6:9, :D].set(dir_emb))
  W1p = jnp.zeros((L, DP, HID), _f32).at[:, :D, :2 * D].set(W1)
  selfc = (jnp.zeros((L, 1, DP), _f32)
           .at[:, 0, :D].set(bond_emb[:, 4, :] + dir_emb[:, 0, :]))
  b1p = jnp.zeros((L, HID), _f32).at[:, :2 * D].set(b1)
  W2p = jnp.zeros((L, HID, DP), _f32).at[:, :2 * D, :D].set(W2)
  b2p = jnp.zeros((L, DP), _f32).at[:, :D].set(b2)
  gmp = jnp.zeros((L, DP), _f32).at[:, :D].set(gamma)
  btp = jnp.zeros((L, DP), _f32).at[:, :D].set(beta)

  pad = EP - E
  srcp = jnp.concatenate([src, jnp.zeros((pad,), _i32)])
  dstp = jnp.concatenate([dst, jnp.full((pad,), DUMMY, _i32)])
  btp_e = jnp.concatenate([bt, jnp.zeros((pad,), _i32)])
  bd6_e = jnp.concatenate([bdt + 6, jnp.full((pad,), 6, _i32)])
  src_s = srcp.reshape(16, NCS, CH)
  dst_s = dstp.reshape(16, NCS, CH)
  bt_s = btp_e.reshape(16, NCS, CH)
  bd6_s = bd6_e.reshape(16, NCS, CH)
  an_r = an.reshape(NB, 1, BS)
  ch_r = ch.reshape(NB, 1, BS)
  gid_r = gid.reshape(NB, 1, BS)
  zr = jnp.zeros((RPT, DS), _f32)
  T = jnp.eye(ET, DS, dtype=_f32)

  ha, hb, hc = _embed_call(an_r, ch_r, emb)
  C2a = _spmm_b_call(T, bt_s, dst_s, zr, stage_table=True)
  C2b = _spmm_b_call(T, bd6_s, dst_s, zr, stage_table=True)
  for l in range(L):
    PA = _spmm_a_call(ha, hb, src_s, dst_s, zr)
    PB = _spmm_b_call(hc, src_s, dst_s, zr)
    h2, st = _mlp_call(PA, PB, ha, hb, hc, C2a, C2b, Et[l], selfc[l], W1p[l],
                       b1p[l].reshape(1, HID), W2p[l], b2p[l].reshape(1, DP))
    ha, hb, hc = _bn_call(h2, st, gmp[l].reshape(1, DP), btp[l].reshape(1, DP),
                          l < L - 1)
  out = _read_call(gid_r, ha, hb, hc)
  return out[:, :D]
